# baseline (speedup 1.0000x reference)
import functools

import jax
import jax.numpy as jnp
from jax import lax
from jax.experimental import pallas as pl
from jax.experimental.pallas import tpu as pltpu
from jax.experimental.pallas import tpu_sc as plsc

NUM_CLASSES = 256
FEATURE_DIM = 512
N_ROWS = 32768
EPS = 1e-08

TC_ROWS = 26624
TC_BLK = 2048
SC_ROWS = N_ROWS - TC_ROWS

N_RG = 8
N_CG = 4
N_TILES = N_RG * N_CG
ROWS_PT = SC_ROWS // N_RG
COLS_PT = FEATURE_DIM // N_CG
CHUNK = 128
N_CHUNKS = ROWS_PT // CHUNK
HIST_PT = N_ROWS // N_TILES
LANES = 16
CNTW = 16


def _bcast_lane(vec, j):
    dnums = lax.GatherDimensionNumbers(
        offset_dims=(), collapsed_slice_dims=(0,), start_index_map=(0,))
    idx = jnp.full((LANES, 1), j, jnp.int32)
    return lax.gather(vec, idx, dnums, (1,),
                      mode=lax.GatherScatterMode.PROMISE_IN_BOUNDS)


def _sc_body(feat_hbm, lab_hbm, zacc_hbm, zhist_hbm,
             out_sums, out_cnts,
             lab_v, labh_v, acc_v, hist_v, buf0_v, buf1_v, sem0, sem1):
    cid = lax.axis_index("c")
    sid = lax.axis_index("s")
    rg = sid % N_RG
    cg = cid * 2 + sid // N_RG
    wid = rg * N_CG + cg
    row0 = TC_ROWS + rg * ROWS_PT
    col0 = cg * COLS_PT

    pltpu.sync_copy(lab_hbm.at[pl.ds(row0, ROWS_PT)], lab_v)
    pltpu.sync_copy(lab_hbm.at[pl.ds(wid * HIST_PT, HIST_PT)], labh_v)
    pltpu.sync_copy(zacc_hbm, acc_v)
    pltpu.sync_copy(zhist_hbm, hist_v)

    col_iotas = [jnp.arange(LANES, dtype=jnp.int32) + (c * LANES)
                 for c in range(COLS_PT // LANES)]
    lane_iota = jnp.arange(LANES, dtype=jnp.int32)
    ones16 = jnp.ones((LANES,), jnp.float32)

    bufs = (buf0_v, buf1_v)
    sems = (sem0, sem1)

    def start_fetch(g, b):
        src = feat_hbm.at[pl.ds(row0 + g * CHUNK, CHUNK), pl.ds(col0, COLS_PT)]
        pltpu.make_async_copy(src, bufs[b], sems[b]).start()

    def wait_fetch(g, b):
        src = feat_hbm.at[pl.ds(row0 + g * CHUNK, CHUNK), pl.ds(col0, COLS_PT)]
        pltpu.make_async_copy(src, bufs[b], sems[b]).wait()

    start_fetch(0, 0)

    @plsc.parallel_loop(0, HIST_PT // LANES, unroll=4)
    def hist_body(r16):
        labels16 = labh_v[pl.ds(r16 * LANES, LANES)]
        plsc.addupdate_scatter(hist_v, [labels16, lane_iota], ones16)

    def chunk_body(g, b, buf):
        wait_fetch(g, b)

        @pl.when(g + 1 < N_CHUNKS)
        def _():
            start_fetch(g + 1, 1 - b)

        @plsc.parallel_loop(0, CHUNK // LANES, unroll=4)
        def row_body(r16):
            labels16 = lab_v[pl.ds(g * CHUNK + r16 * LANES, LANES)]
            for j in range(LANES):
                lbl = _bcast_lane(labels16, j)
                r = r16 * LANES + j
                for c in range(COLS_PT // LANES):
                    data = buf[r, pl.ds(c * LANES, LANES)]
                    plsc.addupdate_scatter(acc_v, [lbl, col_iotas[c]], data)

    def two_chunks(gg, carry):
        chunk_body(2 * gg, 0, buf0_v)
        chunk_body(2 * gg + 1, 1, buf1_v)
        return carry

    lax.fori_loop(0, N_CHUNKS // 2, two_chunks, 0)

    pltpu.sync_copy(acc_v, out_sums.at[rg, :, pl.ds(col0, COLS_PT)])
    pltpu.sync_copy(hist_v, out_cnts.at[wid])


def _make_sc_kernel():
    mesh = plsc.VectorSubcoreMesh(core_axis_name="c", subcore_axis_name="s")
    return pl.kernel(
        _sc_body,
        out_type=[
            jax.ShapeDtypeStruct((N_RG, NUM_CLASSES, FEATURE_DIM), jnp.float32),
            jax.ShapeDtypeStruct((N_TILES, NUM_CLASSES, CNTW), jnp.float32),
        ],
        mesh=mesh,
        compiler_params=pltpu.CompilerParams(needs_layout_passes=False),
        scratch_types=[
            pltpu.VMEM((ROWS_PT,), jnp.int32),
            pltpu.VMEM((HIST_PT,), jnp.int32),
            pltpu.VMEM((NUM_CLASSES, COLS_PT), jnp.float32),
            pltpu.VMEM((NUM_CLASSES, CNTW), jnp.float32),
            pltpu.VMEM((CHUNK, COLS_PT), jnp.float32),
            pltpu.VMEM((CHUNK, COLS_PT), jnp.float32),
            pltpu.SemaphoreType.DMA,
            pltpu.SemaphoreType.DMA,
        ],
    )


def _tc_partial_body(lab_ref, feat_ref, out_ref):
    i = pl.program_id(0)
    lab = lab_ref[0, 0, :]
    cls = lax.broadcasted_iota(jnp.int32, (NUM_CLASSES, TC_BLK), 0)
    onehot = jnp.where(cls == lab[None, :], 1.0, 0.0
                       ).astype(jnp.bfloat16)
    f = feat_ref[...]
    hi = f.astype(jnp.bfloat16)
    r1 = f - hi.astype(jnp.float32)
    mid = r1.astype(jnp.bfloat16)
    lo = (r1 - mid.astype(jnp.float32)).astype(jnp.bfloat16)
    dn = (((1,), (0,)), ((), ()))
    part = (lax.dot_general(onehot, hi, dn,
                            preferred_element_type=jnp.float32)
            + lax.dot_general(onehot, mid, dn,
                              preferred_element_type=jnp.float32)
            + lax.dot_general(onehot, lo, dn,
                              preferred_element_type=jnp.float32))

    @pl.when(i == 0)
    def _():
        out_ref[...] = part

    @pl.when(i > 0)
    def _():
        out_ref[...] += part


def _tc_partial(labels3d, features):
    nb = TC_ROWS // TC_BLK
    return pl.pallas_call(
        _tc_partial_body,
        grid=(nb,),
        in_specs=[
            pl.BlockSpec((1, 1, TC_BLK), lambda i: (i, 0, 0)),
            pl.BlockSpec((TC_BLK, FEATURE_DIM), lambda i: (i, 0)),
        ],
        out_specs=pl.BlockSpec((NUM_CLASSES, FEATURE_DIM), lambda i: (0, 0)),
        out_shape=jax.ShapeDtypeStruct((NUM_CLASSES, FEATURE_DIM),
                                       jnp.float32),
    )(labels3d, features)


def _loss_body(sums_ref, tcsums_ref, cnts_ref, out_ref):
    sums = jnp.sum(sums_ref[...], axis=0) + tcsums_ref[...]
    cnt = jnp.sum(cnts_ref[...], axis=(0, 2)).reshape(NUM_CLASSES, 1)
    present = cnt > 0.0
    safe = jnp.maximum(cnt, 1.0)
    cent = jnp.where(present, sums / safe, 0.0)

    gram = lax.dot_general(cent, cent, (((1,), (1,)), ((), ())),
                           preferred_element_type=jnp.float32)
    ii = lax.broadcasted_iota(jnp.int32, (NUM_CLASSES, NUM_CLASSES), 0)
    jj = lax.broadcasted_iota(jnp.int32, (NUM_CLASSES, NUM_CLASSES), 1)
    eye = ii == jj
    diag_col = jnp.sum(jnp.where(eye, gram, 0.0), axis=1, keepdims=True)
    diag_row = jnp.sum(jnp.where(eye, gram, 0.0), axis=0, keepdims=True)
    dist_sq = jnp.maximum(diag_col + diag_row - 2.0 * gram, 0.0)

    pres_f = jnp.where(present, 1.0, 0.0)
    pres_mat = lax.dot_general(pres_f, pres_f, (((1,), (1,)), ((), ())),
                               preferred_element_type=jnp.float32)
    valid = (ii < jj) & (pres_mat > 0.5)
    safe_sq = jnp.where(valid, dist_sq, 1.0)
    dist = jnp.sqrt(safe_sq) * (1.0 / 16.0)
    terms = jnp.where(valid, jnp.exp(-(dist + EPS)), 0.0)
    out_ref[...] = jnp.reshape(jnp.sum(terms), (1, 1))


def _tc_loss(sums8, tcsums, counts):
    return pl.pallas_call(
        _loss_body,
        out_shape=jax.ShapeDtypeStruct((1, 1), jnp.float32),
    )(sums8, tcsums, counts)


def kernel(features, labels):
    labels = labels.astype(jnp.int32)
    zacc = jnp.zeros((NUM_CLASSES, COLS_PT), jnp.float32)
    zhist = jnp.zeros((NUM_CLASSES, CNTW), jnp.float32)
    sums8, counts = _make_sc_kernel()(features, labels, zacc, zhist)
    lab3d = labels.reshape(N_ROWS // TC_BLK, 1, TC_BLK)
    tcsums = _tc_partial(lab3d, features)
    loss = _tc_loss(sums8, tcsums, counts)
    return loss[0, 0]


# speedup vs baseline: 1.0076x; 1.0076x over previous
import functools

import jax
import jax.numpy as jnp
from jax import lax
from jax.experimental import pallas as pl
from jax.experimental.pallas import tpu as pltpu
from jax.experimental.pallas import tpu_sc as plsc

NUM_CLASSES = 256
FEATURE_DIM = 512
N_ROWS = 32768
EPS = 1e-08

TC_ROWS = 28672
TC_BLK = 2048
SC_ROWS = N_ROWS - TC_ROWS

N_RG = 8
N_CG = 4
N_TILES = N_RG * N_CG
ROWS_PT = SC_ROWS // N_RG
COLS_PT = FEATURE_DIM // N_CG
CHUNK = 128
N_CHUNKS = ROWS_PT // CHUNK
HIST_PT = N_ROWS // N_TILES
LANES = 16
CNTW = 16


def _bcast_lane(vec, j):
    dnums = lax.GatherDimensionNumbers(
        offset_dims=(), collapsed_slice_dims=(0,), start_index_map=(0,))
    idx = jnp.full((LANES, 1), j, jnp.int32)
    return lax.gather(vec, idx, dnums, (1,),
                      mode=lax.GatherScatterMode.PROMISE_IN_BOUNDS)


def _sc_body(feat_hbm, lab_hbm, zacc_hbm, zhist_hbm,
             out_sums, out_cnts,
             lab_v, labh_v, acc_v, hist_v, buf0_v, buf1_v, sem0, sem1):
    cid = lax.axis_index("c")
    sid = lax.axis_index("s")
    rg = sid % N_RG
    cg = cid * 2 + sid // N_RG
    wid = rg * N_CG + cg
    row0 = TC_ROWS + rg * ROWS_PT
    col0 = cg * COLS_PT

    pltpu.sync_copy(lab_hbm.at[pl.ds(row0, ROWS_PT)], lab_v)
    pltpu.sync_copy(lab_hbm.at[pl.ds(wid * HIST_PT, HIST_PT)], labh_v)
    pltpu.sync_copy(zacc_hbm, acc_v)
    pltpu.sync_copy(zhist_hbm, hist_v)

    col_iotas = [jnp.arange(LANES, dtype=jnp.int32) + (c * LANES)
                 for c in range(COLS_PT // LANES)]
    lane_iota = jnp.arange(LANES, dtype=jnp.int32)
    ones16 = jnp.ones((LANES,), jnp.float32)

    bufs = (buf0_v, buf1_v)
    sems = (sem0, sem1)

    def start_fetch(g, b):
        src = feat_hbm.at[pl.ds(row0 + g * CHUNK, CHUNK), pl.ds(col0, COLS_PT)]
        pltpu.make_async_copy(src, bufs[b], sems[b]).start()

    def wait_fetch(g, b):
        src = feat_hbm.at[pl.ds(row0 + g * CHUNK, CHUNK), pl.ds(col0, COLS_PT)]
        pltpu.make_async_copy(src, bufs[b], sems[b]).wait()

    start_fetch(0, 0)

    @plsc.parallel_loop(0, HIST_PT // LANES, unroll=4)
    def hist_body(r16):
        labels16 = labh_v[pl.ds(r16 * LANES, LANES)]
        plsc.addupdate_scatter(hist_v, [labels16, lane_iota], ones16)

    def chunk_body(g, b, buf):
        wait_fetch(g, b)

        @pl.when(g + 1 < N_CHUNKS)
        def _():
            start_fetch(g + 1, 1 - b)

        @plsc.parallel_loop(0, CHUNK // LANES, unroll=4)
        def row_body(r16):
            labels16 = lab_v[pl.ds(g * CHUNK + r16 * LANES, LANES)]
            for j in range(LANES):
                lbl = _bcast_lane(labels16, j)
                r = r16 * LANES + j
                for c in range(COLS_PT // LANES):
                    data = buf[r, pl.ds(c * LANES, LANES)]
                    plsc.addupdate_scatter(acc_v, [lbl, col_iotas[c]], data)

    def two_chunks(gg, carry):
        chunk_body(2 * gg, 0, buf0_v)
        chunk_body(2 * gg + 1, 1, buf1_v)
        return carry

    lax.fori_loop(0, N_CHUNKS // 2, two_chunks, 0)

    pltpu.sync_copy(acc_v, out_sums.at[rg, :, pl.ds(col0, COLS_PT)])
    pltpu.sync_copy(hist_v, out_cnts.at[wid])


def _make_sc_kernel():
    mesh = plsc.VectorSubcoreMesh(core_axis_name="c", subcore_axis_name="s")
    return pl.kernel(
        _sc_body,
        out_type=[
            jax.ShapeDtypeStruct((N_RG, NUM_CLASSES, FEATURE_DIM), jnp.float32),
            jax.ShapeDtypeStruct((N_TILES, NUM_CLASSES, CNTW), jnp.float32),
        ],
        mesh=mesh,
        compiler_params=pltpu.CompilerParams(needs_layout_passes=False),
        scratch_types=[
            pltpu.VMEM((ROWS_PT,), jnp.int32),
            pltpu.VMEM((HIST_PT,), jnp.int32),
            pltpu.VMEM((NUM_CLASSES, COLS_PT), jnp.float32),
            pltpu.VMEM((NUM_CLASSES, CNTW), jnp.float32),
            pltpu.VMEM((CHUNK, COLS_PT), jnp.float32),
            pltpu.VMEM((CHUNK, COLS_PT), jnp.float32),
            pltpu.SemaphoreType.DMA,
            pltpu.SemaphoreType.DMA,
        ],
    )


def _tc_partial_body(lab_ref, feat_ref, out_ref):
    i = pl.program_id(0)
    lab = lab_ref[0, 0, :]
    cls = lax.broadcasted_iota(jnp.int32, (NUM_CLASSES, TC_BLK), 0)
    onehot = jnp.where(cls == lab[None, :], 1.0, 0.0
                       ).astype(jnp.bfloat16)
    f = feat_ref[...]
    hi = f.astype(jnp.bfloat16)
    r1 = f - hi.astype(jnp.float32)
    mid = r1.astype(jnp.bfloat16)
    lo = (r1 - mid.astype(jnp.float32)).astype(jnp.bfloat16)
    dn = (((1,), (0,)), ((), ()))
    part = (lax.dot_general(onehot, hi, dn,
                            preferred_element_type=jnp.float32)
            + lax.dot_general(onehot, mid, dn,
                              preferred_element_type=jnp.float32)
            + lax.dot_general(onehot, lo, dn,
                              preferred_element_type=jnp.float32))

    @pl.when(i == 0)
    def _():
        out_ref[...] = part

    @pl.when(i > 0)
    def _():
        out_ref[...] += part


def _tc_partial(labels3d, features):
    nb = TC_ROWS // TC_BLK
    return pl.pallas_call(
        _tc_partial_body,
        grid=(nb,),
        in_specs=[
            pl.BlockSpec((1, 1, TC_BLK), lambda i: (i, 0, 0)),
            pl.BlockSpec((TC_BLK, FEATURE_DIM), lambda i: (i, 0)),
        ],
        out_specs=pl.BlockSpec((NUM_CLASSES, FEATURE_DIM), lambda i: (0, 0)),
        out_shape=jax.ShapeDtypeStruct((NUM_CLASSES, FEATURE_DIM),
                                       jnp.float32),
    )(labels3d, features)


def _loss_body(sums_ref, tcsums_ref, cnts_ref, out_ref):
    sums = jnp.sum(sums_ref[...], axis=0) + tcsums_ref[...]
    cnt = jnp.sum(cnts_ref[...], axis=(0, 2)).reshape(NUM_CLASSES, 1)
    present = cnt > 0.0
    safe = jnp.maximum(cnt, 1.0)
    cent = jnp.where(present, sums / safe, 0.0)

    gram = lax.dot_general(cent, cent, (((1,), (1,)), ((), ())),
                           preferred_element_type=jnp.float32)
    ii = lax.broadcasted_iota(jnp.int32, (NUM_CLASSES, NUM_CLASSES), 0)
    jj = lax.broadcasted_iota(jnp.int32, (NUM_CLASSES, NUM_CLASSES), 1)
    eye = ii == jj
    diag_col = jnp.sum(jnp.where(eye, gram, 0.0), axis=1, keepdims=True)
    diag_row = jnp.sum(jnp.where(eye, gram, 0.0), axis=0, keepdims=True)
    dist_sq = jnp.maximum(diag_col + diag_row - 2.0 * gram, 0.0)

    pres_f = jnp.where(present, 1.0, 0.0)
    pres_mat = lax.dot_general(pres_f, pres_f, (((1,), (1,)), ((), ())),
                               preferred_element_type=jnp.float32)
    valid = (ii < jj) & (pres_mat > 0.5)
    safe_sq = jnp.where(valid, dist_sq, 1.0)
    dist = jnp.sqrt(safe_sq) * (1.0 / 16.0)
    terms = jnp.where(valid, jnp.exp(-(dist + EPS)), 0.0)
    out_ref[...] = jnp.reshape(jnp.sum(terms), (1, 1))


def _tc_loss(sums8, tcsums, counts):
    return pl.pallas_call(
        _loss_body,
        out_shape=jax.ShapeDtypeStruct((1, 1), jnp.float32),
    )(sums8, tcsums, counts)


def kernel(features, labels):
    labels = labels.astype(jnp.int32)
    zacc = jnp.zeros((NUM_CLASSES, COLS_PT), jnp.float32)
    zhist = jnp.zeros((NUM_CLASSES, CNTW), jnp.float32)
    sums8, counts = _make_sc_kernel()(features, labels, zacc, zhist)
    lab3d = labels.reshape(N_ROWS // TC_BLK, 1, TC_BLK)
    tcsums = _tc_partial(lab3d, features)
    loss = _tc_loss(sums8, tcsums, counts)
    return loss[0, 0]


# speedup vs baseline: 1.0161x; 1.0084x over previous
import functools

import jax
import jax.numpy as jnp
from jax import lax
from jax.experimental import pallas as pl
from jax.experimental.pallas import tpu as pltpu
from jax.experimental.pallas import tpu_sc as plsc

NUM_CLASSES = 256
FEATURE_DIM = 512
N_ROWS = 32768
EPS = 1e-08

TC_ROWS = 26624
TC_BLK = 2048
SC_ROWS = N_ROWS - TC_ROWS

N_RG = 8
N_CG = 4
N_TILES = N_RG * N_CG
ROWS_PT = SC_ROWS // N_RG
COLS_PT = FEATURE_DIM // N_CG
CHUNK = 128
N_CHUNKS = ROWS_PT // CHUNK
HIST_PT = N_ROWS // N_TILES
LANES = 16
CNTW = 16


def _bcast_lane(vec, j):
    dnums = lax.GatherDimensionNumbers(
        offset_dims=(), collapsed_slice_dims=(0,), start_index_map=(0,))
    idx = jnp.full((LANES, 1), j, jnp.int32)
    return lax.gather(vec, idx, dnums, (1,),
                      mode=lax.GatherScatterMode.PROMISE_IN_BOUNDS)


def _sc_body(feat_hbm, lab_hbm, zacc_hbm, zhist_hbm,
             out_sums, out_cnts,
             lab_v, labh_v, acc_v, hist_v, buf0_v, buf1_v, sem0, sem1):
    cid = lax.axis_index("c")
    sid = lax.axis_index("s")
    rg = sid % N_RG
    cg = cid * 2 + sid // N_RG
    wid = rg * N_CG + cg
    row0 = TC_ROWS + rg * ROWS_PT
    col0 = cg * COLS_PT

    pltpu.sync_copy(lab_hbm.at[pl.ds(row0, ROWS_PT)], lab_v)
    pltpu.sync_copy(lab_hbm.at[pl.ds(wid * HIST_PT, HIST_PT)], labh_v)
    pltpu.sync_copy(zacc_hbm, acc_v)
    pltpu.sync_copy(zhist_hbm, hist_v)

    col_iotas = [jnp.arange(LANES, dtype=jnp.int32) + (c * LANES)
                 for c in range(COLS_PT // LANES)]
    lane_iota = jnp.arange(LANES, dtype=jnp.int32)
    ones16 = jnp.ones((LANES,), jnp.float32)

    bufs = (buf0_v, buf1_v)
    sems = (sem0, sem1)

    def start_fetch(g, b):
        src = feat_hbm.at[pl.ds(row0 + g * CHUNK, CHUNK), pl.ds(col0, COLS_PT)]
        pltpu.make_async_copy(src, bufs[b], sems[b]).start()

    def wait_fetch(g, b):
        src = feat_hbm.at[pl.ds(row0 + g * CHUNK, CHUNK), pl.ds(col0, COLS_PT)]
        pltpu.make_async_copy(src, bufs[b], sems[b]).wait()

    start_fetch(0, 0)

    @plsc.parallel_loop(0, HIST_PT // LANES, unroll=4)
    def hist_body(r16):
        labels16 = labh_v[pl.ds(r16 * LANES, LANES)]
        plsc.addupdate_scatter(hist_v, [labels16, lane_iota], ones16)

    def chunk_body(g, b, buf):
        wait_fetch(g, b)

        @pl.when(g + 1 < N_CHUNKS)
        def _():
            start_fetch(g + 1, 1 - b)

        @plsc.parallel_loop(0, CHUNK // LANES, unroll=4)
        def row_body(r16):
            labels16 = lab_v[pl.ds(g * CHUNK + r16 * LANES, LANES)]
            for j in range(LANES):
                lbl = _bcast_lane(labels16, j)
                r = r16 * LANES + j
                for c in range(COLS_PT // LANES):
                    data = buf[r, pl.ds(c * LANES, LANES)]
                    plsc.addupdate_scatter(acc_v, [lbl, col_iotas[c]], data)

    def two_chunks(gg, carry):
        chunk_body(2 * gg, 0, buf0_v)
        chunk_body(2 * gg + 1, 1, buf1_v)
        return carry

    lax.fori_loop(0, N_CHUNKS // 2, two_chunks, 0)

    pltpu.sync_copy(acc_v, out_sums.at[rg, :, pl.ds(col0, COLS_PT)])
    pltpu.sync_copy(hist_v, out_cnts.at[wid])


def _make_sc_kernel():
    mesh = plsc.VectorSubcoreMesh(core_axis_name="c", subcore_axis_name="s")
    return pl.kernel(
        _sc_body,
        out_type=[
            jax.ShapeDtypeStruct((N_RG, NUM_CLASSES, FEATURE_DIM), jnp.float32),
            jax.ShapeDtypeStruct((N_TILES, NUM_CLASSES, CNTW), jnp.float32),
        ],
        mesh=mesh,
        compiler_params=pltpu.CompilerParams(needs_layout_passes=False),
        scratch_types=[
            pltpu.VMEM((ROWS_PT,), jnp.int32),
            pltpu.VMEM((HIST_PT,), jnp.int32),
            pltpu.VMEM((NUM_CLASSES, COLS_PT), jnp.float32),
            pltpu.VMEM((NUM_CLASSES, CNTW), jnp.float32),
            pltpu.VMEM((CHUNK, COLS_PT), jnp.float32),
            pltpu.VMEM((CHUNK, COLS_PT), jnp.float32),
            pltpu.SemaphoreType.DMA,
            pltpu.SemaphoreType.DMA,
        ],
    )


def _tc_partial_body(lab_ref, feat_ref, out_ref):
    i = pl.program_id(0)
    lab = lab_ref[0, 0, :]
    cls = lax.broadcasted_iota(jnp.int32, (NUM_CLASSES, TC_BLK), 0)
    onehot = jnp.where(cls == lab[None, :], 1.0, 0.0)
    part = lax.dot_general(onehot, feat_ref[...], (((1,), (0,)), ((), ())),
                           preferred_element_type=jnp.float32)

    @pl.when(i == 0)
    def _():
        out_ref[...] = part

    @pl.when(i > 0)
    def _():
        out_ref[...] += part


def _tc_partial(labels3d, features):
    nb = TC_ROWS // TC_BLK
    return pl.pallas_call(
        _tc_partial_body,
        grid=(nb,),
        in_specs=[
            pl.BlockSpec((1, 1, TC_BLK), lambda i: (i, 0, 0)),
            pl.BlockSpec((TC_BLK, FEATURE_DIM), lambda i: (i, 0)),
        ],
        out_specs=pl.BlockSpec((NUM_CLASSES, FEATURE_DIM), lambda i: (0, 0)),
        out_shape=jax.ShapeDtypeStruct((NUM_CLASSES, FEATURE_DIM),
                                       jnp.float32),
    )(labels3d, features)


def _loss_body(sums_ref, tcsums_ref, cnts_ref, out_ref):
    sums = jnp.sum(sums_ref[...], axis=0) + tcsums_ref[...]
    cnt = jnp.sum(cnts_ref[...], axis=(0, 2)).reshape(NUM_CLASSES, 1)
    present = cnt > 0.0
    safe = jnp.maximum(cnt, 1.0)
    cent = jnp.where(present, sums / safe, 0.0)

    gram = lax.dot_general(cent, cent, (((1,), (1,)), ((), ())),
                           preferred_element_type=jnp.float32)
    ii = lax.broadcasted_iota(jnp.int32, (NUM_CLASSES, NUM_CLASSES), 0)
    jj = lax.broadcasted_iota(jnp.int32, (NUM_CLASSES, NUM_CLASSES), 1)
    eye = ii == jj
    diag_col = jnp.sum(jnp.where(eye, gram, 0.0), axis=1, keepdims=True)
    diag_row = jnp.sum(jnp.where(eye, gram, 0.0), axis=0, keepdims=True)
    dist_sq = jnp.maximum(diag_col + diag_row - 2.0 * gram, 0.0)

    pres_f = jnp.where(present, 1.0, 0.0)
    pres_mat = lax.dot_general(pres_f, pres_f, (((1,), (1,)), ((), ())),
                               preferred_element_type=jnp.float32)
    valid = (ii < jj) & (pres_mat > 0.5)
    safe_sq = jnp.where(valid, dist_sq, 1.0)
    dist = jnp.sqrt(safe_sq) * (1.0 / 16.0)
    terms = jnp.where(valid, jnp.exp(-(dist + EPS)), 0.0)
    out_ref[...] = jnp.reshape(jnp.sum(terms), (1, 1))


def _tc_loss(sums8, tcsums, counts):
    return pl.pallas_call(
        _loss_body,
        out_shape=jax.ShapeDtypeStruct((1, 1), jnp.float32),
    )(sums8, tcsums, counts)


def kernel(features, labels):
    labels = labels.astype(jnp.int32)
    zacc = jnp.zeros((NUM_CLASSES, COLS_PT), jnp.float32)
    zhist = jnp.zeros((NUM_CLASSES, CNTW), jnp.float32)
    sums8, counts = _make_sc_kernel()(features, labels, zacc, zhist)
    lab3d = labels.reshape(N_ROWS // TC_BLK, 1, TC_BLK)
    tcsums = _tc_partial(lab3d, features)
    loss = _tc_loss(sums8, tcsums, counts)
    return loss[0, 0]
